# baseline (speedup 1.0000x reference)
import functools

import jax
import jax.numpy as jnp
from jax import lax
from jax.experimental import pallas as pl
from jax.experimental.pallas import tpu as pltpu
from jax.experimental.pallas import tpu_sc as plsc

_N = 10000
_E = 320000
_DHID = 128

_NC = 2
_NS = 16
_NW = _NC * _NS
_EPW = _E // _NW
_CH = 128
_EPW_PAD = 10240
_EPAD = _NW * _EPW_PAD
_NCHUNKS = _EPW_PAD // _CH
_HCHUNKS = _NCHUNKS // 2
_NPAD = 10240
_RPT = _NPAD // _NS
_CHD = 10000
_NCHUNKS_D = _EPW // _CHD


def _make_prop():
    mesh = plsc.VectorSubcoreMesh(core_axis_name="c", subcore_axis_name="s")

    @functools.partial(
        pl.kernel, mesh=mesh,
        out_type=jax.ShapeDtypeStruct((_NC * _NPAD, _DHID), jnp.float32),
        scratch_types=[
            pltpu.VMEM((_HCHUNKS, _CH), jnp.int32),
            pltpu.VMEM((_HCHUNKS, _CH), jnp.int32),
            pltpu.VMEM((_CH, _DHID), jnp.float32),
            pltpu.VMEM((_CH, _DHID), jnp.float32),
            pltpu.VMEM_SHARED((_NPAD, _DHID), jnp.float32),
            pltpu.SemaphoreType.DMA,
            pltpu.SemaphoreType.DMA,
        ],
    )
    def k(src, rows3d, cols3d, zrows, out, ridx, cidx, rbuf0, rbuf1, acc,
          sg0, sg1):
        c = lax.axis_index("c")
        s = lax.axis_index("s")
        wid = s * _NC + c
        bufs = ((rbuf0, sg0), (rbuf1, sg1))

        pltpu.sync_copy(zrows, acc.at[pl.ds(s * _RPT, _RPT)])
        plsc.subcore_barrier()

        def start_gather(b, i):
            rb, sg = bufs[b]
            return pltpu.async_copy(src.at[ridx.at[i]], rb, sg)

        def finish(b, i, start_next):
            rb, sg = bufs[b]
            pltpu.make_async_copy(src.at[ridx.at[i]], rb, sg).wait()
            pltpu.sync_copy(rb, acc.at[cidx.at[i]], add=True)
            if start_next:
                start_gather(b, i + 2)

        for ph in range(_NCHUNKS // _HCHUNKS):
            pltpu.sync_copy(rows3d.at[wid, pl.ds(ph * _HCHUNKS, _HCHUNKS)],
                            ridx)
            pltpu.sync_copy(cols3d.at[wid, pl.ds(ph * _HCHUNKS, _HCHUNKS)],
                            cidx)
            start_gather(0, 0)
            start_gather(1, 1)

            def pair(kk, carry):
                finish(0, 2 * kk, True)
                finish(1, 2 * kk + 1, True)
                return carry

            lax.fori_loop(0, _HCHUNKS // 2 - 1, pair, 0)
            finish(0, _HCHUNKS - 2, False)
            finish(1, _HCHUNKS - 1, False)
        plsc.subcore_barrier()

        pltpu.sync_copy(acc.at[pl.ds(s * _RPT, _RPT)],
                        out.at[pl.ds(c * _NPAD + s * _RPT, _RPT)])

    return k


def _make_deg():
    mesh = plsc.VectorSubcoreMesh(core_axis_name="c", subcore_axis_name="s")

    @functools.partial(
        pl.kernel, mesh=mesh,
        out_type=jax.ShapeDtypeStruct((_NW, _NPAD), jnp.float32),
        scratch_types=[
            pltpu.VMEM((_CHD,), jnp.int32),
            pltpu.VMEM((_NPAD,), jnp.float32),
        ],
        compiler_params=pltpu.CompilerParams(needs_layout_passes=False),
    )
    def k(cols, out, cidx, hist):
        c = lax.axis_index("c")
        s = lax.axis_index("s")
        wid = s * _NC + c

        z16 = jnp.zeros((16,), jnp.float32)

        def zero(i, carry):
            hist[pl.ds(i * 16, 16)] = z16
            return carry

        lax.fori_loop(0, _NPAD // 16, zero, 0)

        def chunk(i, carry):
            base = wid * _EPW + i * _CHD
            pltpu.sync_copy(cols.at[pl.ds(base, _CHD)], cidx)

            def vec(j, carry2):
                idx16 = cidx[pl.ds(j * 16, 16)]
                cnt, last = plsc.scan_count(idx16)
                plsc.addupdate_scatter(
                    hist, [idx16], cnt.astype(jnp.float32), mask=last)
                return carry2

            return lax.fori_loop(0, _CHD // 16, vec, carry)

        lax.fori_loop(0, _NCHUNKS_D, chunk, 0)
        pltpu.sync_copy(hist, out.at[wid])

    return k


_prop128 = _make_prop()
_deg_pass = _make_deg()


def _dinv_from(deg_part_ref):
    ones32 = jnp.ones((_NW, 1), jnp.float32)
    deg = lax.dot_general(deg_part_ref[...], ones32,
                          (((0,), (0,)), ((), ())),
                          preferred_element_type=jnp.float32)
    return lax.rsqrt(deg[: _N, :] + 1.0)


def _tc1_body(dp, f, w, g1):
    dinv = _dinv_from(dp)
    g1[...] = jnp.dot(f[...], w[...], preferred_element_type=jnp.float32) * dinv


def _tc2_body(dp, scat1, g1, b1, y):
    dinv = _dinv_from(dp)
    agg = scat1[: _N, :] + scat1[_NPAD : _NPAD + _N, :] + g1[...]
    y[...] = jnp.maximum(agg * dinv + b1[...], 0.0) * dinv


def _tc3_body(dp, scat2, y, w2, b2, out):
    dinv = _dinv_from(dp)
    z = (scat2[: _N, :] + scat2[_NPAD : _NPAD + _N, :] + y[...]) * dinv
    out[...] = jnp.dot(z, w2[...], preferred_element_type=jnp.float32) + b2[...]


def kernel(features, edges, edges2, edge_features, W1, b1, W2, b2):
    del edges2, edge_features
    rows = edges[0]
    cols = edges[1]

    npad_e = _EPAD - _E
    rows3d = jnp.concatenate(
        [rows, jnp.zeros((npad_e,), jnp.int32)]).reshape(_NW, _NCHUNKS, _CH)
    cols3d = jnp.concatenate(
        [cols, jnp.full((npad_e,), _N + 16, jnp.int32)]).reshape(
            _NW, _NCHUNKS, _CH)

    zeros128 = jnp.zeros((_RPT, _DHID), jnp.float32)
    b1_2d = b1.reshape(1, _DHID)
    b2_2d = b2.reshape(1, 3)

    deg_part = _deg_pass(cols)

    g1 = pl.pallas_call(
        _tc1_body,
        out_shape=jax.ShapeDtypeStruct((_N, _DHID), jnp.float32),
    )(deg_part, features, W1)

    scat1 = _prop128(g1, rows3d, cols3d, zeros128)

    y = pl.pallas_call(
        _tc2_body,
        out_shape=jax.ShapeDtypeStruct((_N, _DHID), jnp.float32),
    )(deg_part, scat1, g1, b1_2d)

    scat2 = _prop128(y, rows3d, cols3d, zeros128)

    out = pl.pallas_call(
        _tc3_body,
        out_shape=jax.ShapeDtypeStruct((_N, 3), jnp.float32),
    )(deg_part, scat2, y, W2, b2_2d)

    return out


# speedup vs baseline: 1.6090x; 1.6090x over previous
import functools

import jax
import jax.numpy as jnp
from jax import lax
from jax.experimental import pallas as pl
from jax.experimental.pallas import tpu as pltpu
from jax.experimental.pallas import tpu_sc as plsc

_N = 10000
_E = 320000
_DHID = 128

_NC = 2
_NS = 16
_NW = _NC * _NS
_EPW = _E // _NW
_CH = 80
_NCHUNKS = _EPW // _CH
_NPAD = 10240
_RPT = _NPAD // _NS
_CHD = 10000
_NCHUNKS_D = _EPW // _CHD


def _make_prop(d, tc_tiling=True):
    mesh = plsc.VectorSubcoreMesh(core_axis_name="c", subcore_axis_name="s")

    @functools.partial(
        pl.kernel, mesh=mesh,
        out_type=jax.ShapeDtypeStruct((_NC * _NPAD, d), jnp.float32),
        scratch_types=[
            pltpu.VMEM((_CH,), jnp.int32),
            pltpu.VMEM((_CH,), jnp.int32),
            pltpu.VMEM((_CH, d), jnp.float32),
            pltpu.VMEM_SHARED((_NPAD, d), jnp.float32),
            pltpu.SemaphoreType.DMA,
        ],
        compiler_params=pltpu.CompilerParams(use_tc_tiling_on_sc=tc_tiling),
    )
    def k(src, rows, cols, zrows, out, ridx, cidx, rbuf, acc, sem):
        c = lax.axis_index("c")
        s = lax.axis_index("s")
        wid = s * _NC + c

        pltpu.sync_copy(zrows, acc.at[pl.ds(s * _RPT, _RPT)])
        plsc.subcore_barrier()

        def chunk(i, carry):
            base = wid * _EPW + i * _CH
            pltpu.sync_copy(rows.at[pl.ds(base, _CH)], ridx)
            pltpu.async_copy(src.at[ridx], rbuf, sem).wait()
            pltpu.sync_copy(cols.at[pl.ds(base, _CH)], cidx)
            pltpu.sync_copy(rbuf, acc.at[cidx], add=True)
            return carry

        lax.fori_loop(0, _NCHUNKS, chunk, 0)
        plsc.subcore_barrier()

        pltpu.sync_copy(acc.at[pl.ds(s * _RPT, _RPT)],
                        out.at[pl.ds(c * _NPAD + s * _RPT, _RPT)])

    return k


def _make_deg():
    mesh = plsc.VectorSubcoreMesh(core_axis_name="c", subcore_axis_name="s")

    @functools.partial(
        pl.kernel, mesh=mesh,
        out_type=jax.ShapeDtypeStruct((_NW, _NPAD), jnp.float32),
        scratch_types=[
            pltpu.VMEM((_CHD,), jnp.int32),
            pltpu.VMEM((_NPAD,), jnp.float32),
        ],
        compiler_params=pltpu.CompilerParams(needs_layout_passes=False),
    )
    def k(cols, out, cidx, hist):
        c = lax.axis_index("c")
        s = lax.axis_index("s")
        wid = s * _NC + c

        z16 = jnp.zeros((16,), jnp.float32)

        def zero(i, carry):
            hist[pl.ds(i * 16, 16)] = z16
            return carry

        lax.fori_loop(0, _NPAD // 16, zero, 0)

        def chunk(i, carry):
            base = wid * _EPW + i * _CHD
            pltpu.sync_copy(cols.at[pl.ds(base, _CHD)], cidx)

            def vec(j, carry2):
                idx16 = cidx[pl.ds(j * 16, 16)]
                cnt, last = plsc.scan_count(idx16)
                plsc.addupdate_scatter(
                    hist, [idx16], cnt.astype(jnp.float32), mask=last)
                return carry2

            return lax.fori_loop(0, _CHD // 16, vec, carry)

        lax.fori_loop(0, _NCHUNKS_D, chunk, 0)
        pltpu.sync_copy(hist, out.at[wid])

    return k


_prop128 = _make_prop(_DHID)
_prop8 = _make_prop(8, tc_tiling=False)
_deg_pass = _make_deg()


def _dinv_from(deg_part_ref):
    ones32 = jnp.ones((_NW, 1), jnp.float32)
    deg = lax.dot_general(deg_part_ref[...], ones32,
                          (((0,), (0,)), ((), ())),
                          preferred_element_type=jnp.float32)
    return lax.rsqrt(deg[: _N, :] + 1.0)


def _tc1_body(dp, f, w, g1):
    dinv = _dinv_from(dp)
    g1[...] = jnp.dot(f[...], w[...], preferred_element_type=jnp.float32) * dinv


def _tc2_body(dp, scat1, g1, b1, w2p, g2):
    dinv = _dinv_from(dp)
    agg = scat1[: _N, :] + scat1[_NPAD : _NPAD + _N, :] + g1[...]
    x1 = jnp.maximum(agg * dinv + b1[...], 0.0)
    g2[...] = jnp.dot(x1, w2p[...], preferred_element_type=jnp.float32) * dinv


def _tc3_body(dp, scat2, g2, b2p, out):
    dinv = _dinv_from(dp)
    z = (scat2[: _N, :] + scat2[_NPAD : _NPAD + _N, :] + g2[...]) * dinv
    out[...] = z[:, :3] + b2p[...]


def kernel(features, edges, edges2, edge_features, W1, b1, W2, b2):
    del edges2, edge_features
    rows = edges[0]
    cols = edges[1]

    zeros128 = jnp.zeros((_RPT, _DHID), jnp.float32)
    zeros8 = jnp.zeros((_RPT, 8), jnp.float32)
    b1_2d = b1.reshape(1, _DHID)
    b2_2d = b2.reshape(1, 3)
    w2p = jnp.zeros((_DHID, 8), jnp.float32).at[:, :3].set(W2)

    deg_part = _deg_pass(cols)

    g1 = pl.pallas_call(
        _tc1_body,
        out_shape=jax.ShapeDtypeStruct((_N, _DHID), jnp.float32),
    )(deg_part, features, W1)

    scat1 = _prop128(g1, rows, cols, zeros128)

    g2 = pl.pallas_call(
        _tc2_body,
        out_shape=jax.ShapeDtypeStruct((_N, 8), jnp.float32),
    )(deg_part, scat1, g1, b1_2d, w2p)

    scat2 = _prop8(g2, rows, cols, zeros8)

    out = pl.pallas_call(
        _tc3_body,
        out_shape=jax.ShapeDtypeStruct((_N, 3), jnp.float32),
    )(deg_part, scat2, g2, b2_2d)

    return out


# speedup vs baseline: 4.1388x; 2.5723x over previous
import functools

import jax
import jax.numpy as jnp
from jax import lax
from jax.experimental import pallas as pl
from jax.experimental.pallas import tpu as pltpu
from jax.experimental.pallas import tpu_sc as plsc

_N = 10000
_E = 320000
_DHID = 128

_NC = 2
_NS = 16
_NW = _NC * _NS
_EPW = _E // _NW
_CH = 80
_NCHUNKS = _EPW // _CH
_NPAD = 10240
_RPT = _NPAD // _NS
_CHD = 10000
_NCHUNKS_D = _EPW // _CHD


def _make_prop(d, ch, tc_tiling=True):
    nch = _EPW // ch
    assert nch * ch == _EPW and nch % 2 == 1 and ch % 16 == 0
    nblk = (nch - 1) // 2
    mesh = plsc.VectorSubcoreMesh(core_axis_name="c", subcore_axis_name="s")

    @functools.partial(
        pl.kernel, mesh=mesh,
        out_type=jax.ShapeDtypeStruct((_NC * _NPAD, d), jnp.float32),
        scratch_types=[
            pltpu.VMEM((_EPW,), jnp.int32),
            pltpu.VMEM((_EPW,), jnp.int32),
            pltpu.VMEM((ch,), jnp.int32),
            pltpu.VMEM((ch, d), jnp.float32),
            pltpu.VMEM((ch, d), jnp.float32),
            pltpu.VMEM_SHARED((_NPAD, d), jnp.float32),
            pltpu.SemaphoreType.DMA,
            pltpu.SemaphoreType.DMA,
        ],
        compiler_params=pltpu.CompilerParams(use_tc_tiling_on_sc=tc_tiling),
    )
    def k(src, rows, cols, zrows, out, rall, call, cidx, rb0, rb1, acc,
          sem0, sem1):
        c = lax.axis_index("c")
        s = lax.axis_index("s")
        wid = s * _NC + c
        base = wid * _EPW

        rb = (rb0, rb1)
        sems = (sem0, sem1)

        pltpu.async_copy(rows.at[pl.ds(base, _EPW)], rall, sem0)
        pltpu.async_copy(cols.at[pl.ds(base, _EPW)], call, sem1)
        pltpu.sync_copy(zrows, acc.at[pl.ds(s * _RPT, _RPT)])
        pltpu.make_async_copy(rows.at[pl.ds(base, _EPW)], rall, sem0).wait()
        pltpu.make_async_copy(cols.at[pl.ds(base, _EPW)], call, sem1).wait()
        plsc.subcore_barrier()

        def start_gather(i, b):
            pltpu.async_copy(src.at[rall.at[pl.ds(i * ch, ch)]], rb[b],
                             sems[b])

        def wait_gather(i, b):
            pltpu.make_async_copy(src.at[rall.at[pl.ds(i * ch, ch)]], rb[b],
                                  sems[b]).wait()

        def fill_cidx(i):
            def mv(j, carry):
                cidx[pl.ds(j * 16, 16)] = call[pl.ds(i * ch + j * 16, 16)]
                return carry
            lax.fori_loop(0, ch // 16, mv, 0)

        def scatter(b):
            pltpu.sync_copy(rb[b], acc.at[cidx], add=True)

        start_gather(0, 0)

        def blk(g, carry):
            for b, off in ((1, 1), (0, 2)):
                i = 2 * g + off
                start_gather(i, b)
                fill_cidx(i - 1)
                wait_gather(i - 1, 1 - b)
                scatter(1 - b)
            return carry

        lax.fori_loop(0, nblk, blk, 0)

        fill_cidx(nch - 1)
        wait_gather(nch - 1, 0)
        scatter(0)

        plsc.subcore_barrier()
        pltpu.sync_copy(acc.at[pl.ds(s * _RPT, _RPT)],
                        out.at[pl.ds(c * _NPAD + s * _RPT, _RPT)])

    return k


def _make_deg():
    mesh = plsc.VectorSubcoreMesh(core_axis_name="c", subcore_axis_name="s")

    @functools.partial(
        pl.kernel, mesh=mesh,
        out_type=jax.ShapeDtypeStruct((_NW, _NPAD), jnp.float32),
        scratch_types=[
            pltpu.VMEM((_CHD,), jnp.int32),
            pltpu.VMEM((_NPAD,), jnp.float32),
        ],
        compiler_params=pltpu.CompilerParams(needs_layout_passes=False),
    )
    def k(cols, out, cidx, hist):
        c = lax.axis_index("c")
        s = lax.axis_index("s")
        wid = s * _NC + c

        z16 = jnp.zeros((16,), jnp.float32)

        def zero(i, carry):
            hist[pl.ds(i * 16, 16)] = z16
            return carry

        lax.fori_loop(0, _NPAD // 16, zero, 0)

        def chunk(i, carry):
            base = wid * _EPW + i * _CHD
            pltpu.sync_copy(cols.at[pl.ds(base, _CHD)], cidx)

            def vec(j, carry2):
                idx16 = cidx[pl.ds(j * 16, 16)]
                cnt, last = plsc.scan_count(idx16)
                plsc.addupdate_scatter(
                    hist, [idx16], cnt.astype(jnp.float32), mask=last)
                return carry2

            return lax.fori_loop(0, _CHD // 16, vec, carry)

        lax.fori_loop(0, _NCHUNKS_D, chunk, 0)
        pltpu.sync_copy(hist, out.at[wid])

    return k


_prop128 = _make_prop(_DHID, _CH)
_prop8 = _make_prop(8, 400, tc_tiling=False)
_deg_pass = _make_deg()


def _dinv_from(deg_part_ref):
    ones32 = jnp.ones((_NW, 1), jnp.float32)
    deg = lax.dot_general(deg_part_ref[...], ones32,
                          (((0,), (0,)), ((), ())),
                          preferred_element_type=jnp.float32)
    return lax.rsqrt(deg[: _N, :] + 1.0)


def _tc1_body(dp, f, w, g1):
    dinv = _dinv_from(dp)
    g1[...] = jnp.dot(f[...], w[...], preferred_element_type=jnp.float32) * dinv


def _tc2_body(dp, scat1, g1, b1, w2p, g2):
    dinv = _dinv_from(dp)
    agg = scat1[: _N, :] + scat1[_NPAD : _NPAD + _N, :] + g1[...]
    x1 = jnp.maximum(agg * dinv + b1[...], 0.0)
    g2[...] = jnp.dot(x1, w2p[...], preferred_element_type=jnp.float32) * dinv


def _tc3_body(dp, scat2, g2, b2p, out):
    dinv = _dinv_from(dp)
    z = (scat2[: _N, :] + scat2[_NPAD : _NPAD + _N, :] + g2[...]) * dinv
    out[...] = z[:, :3] + b2p[...]


def kernel(features, edges, edges2, edge_features, W1, b1, W2, b2):
    del edges2, edge_features
    rows = edges[0]
    cols = edges[1]

    zeros128 = jnp.zeros((_RPT, _DHID), jnp.float32)
    zeros8 = jnp.zeros((_RPT, 8), jnp.float32)
    b1_2d = b1.reshape(1, _DHID)
    b2_2d = b2.reshape(1, 3)
    w2p = jnp.zeros((_DHID, 8), jnp.float32).at[:, :3].set(W2)

    deg_part = _deg_pass(cols)

    g1 = pl.pallas_call(
        _tc1_body,
        out_shape=jax.ShapeDtypeStruct((_N, _DHID), jnp.float32),
    )(deg_part, features, W1)

    scat1 = _prop128(g1, rows, cols, zeros128)

    g2 = pl.pallas_call(
        _tc2_body,
        out_shape=jax.ShapeDtypeStruct((_N, 8), jnp.float32),
    )(deg_part, scat1, g1, b1_2d, w2p)

    scat2 = _prop8(g2, rows, cols, zeros8)

    out = pl.pallas_call(
        _tc3_body,
        out_shape=jax.ShapeDtypeStruct((_N, 3), jnp.float32),
    )(deg_part, scat2, g2, b2_2d)

    return out


# speedup vs baseline: 4.2072x; 1.0165x over previous
import functools

import jax
import jax.numpy as jnp
from jax import lax
from jax.experimental import pallas as pl
from jax.experimental.pallas import tpu as pltpu
from jax.experimental.pallas import tpu_sc as plsc

_N = 10000
_E = 320000
_DHID = 128

_NC = 2
_NS = 16
_NW = _NC * _NS
_EPW = _E // _NW
_CH = 80
_NCHUNKS = _EPW // _CH
_NPAD = 10240
_RPT = _NPAD // _NS
_CHD = 10000
_NCHUNKS_D = _EPW // _CHD


def _make_prop(d, ch, tc_tiling=True):
    nch = _EPW // ch
    assert nch * ch == _EPW and nch % 2 == 1 and ch % 16 == 0
    nblk = (nch - 1) // 2
    mesh = plsc.VectorSubcoreMesh(core_axis_name="c", subcore_axis_name="s")

    @functools.partial(
        pl.kernel, mesh=mesh,
        out_type=jax.ShapeDtypeStruct((_NC * _NPAD, d), jnp.float32),
        scratch_types=[
            pltpu.VMEM((_EPW,), jnp.int32),
            pltpu.VMEM((_EPW,), jnp.int32),
            pltpu.VMEM((ch,), jnp.int32),
            pltpu.VMEM((ch, d), jnp.float32),
            pltpu.VMEM((ch, d), jnp.float32),
            pltpu.VMEM_SHARED((_NPAD, d), jnp.float32),
            pltpu.SemaphoreType.DMA,
            pltpu.SemaphoreType.DMA,
        ],
        compiler_params=pltpu.CompilerParams(use_tc_tiling_on_sc=tc_tiling),
    )
    def k(src, rows, cols, zrows, out, rall, call, cidx, rb0, rb1, acc,
          sem0, sem1):
        c = lax.axis_index("c")
        s = lax.axis_index("s")
        wid = s * _NC + c
        base = wid * _EPW

        rb = (rb0, rb1)
        sems = (sem0, sem1)

        pltpu.async_copy(rows.at[pl.ds(base, _EPW)], rall, sem0)
        pltpu.async_copy(cols.at[pl.ds(base, _EPW)], call, sem1)
        pltpu.sync_copy(zrows, acc.at[pl.ds(s * _RPT, _RPT)])
        pltpu.make_async_copy(rows.at[pl.ds(base, _EPW)], rall, sem0).wait()
        pltpu.make_async_copy(cols.at[pl.ds(base, _EPW)], call, sem1).wait()
        plsc.subcore_barrier()

        def start_gather(i, b):
            pltpu.async_copy(src.at[rall.at[pl.ds(i * ch, ch)]], rb[b],
                             sems[b])

        def wait_gather(i, b):
            pltpu.make_async_copy(src.at[rall.at[pl.ds(i * ch, ch)]], rb[b],
                                  sems[b]).wait()

        def fill_cidx(i):
            def mv(j, carry):
                cidx[pl.ds(j * 16, 16)] = call[pl.ds(i * ch + j * 16, 16)]
                return carry
            lax.fori_loop(0, ch // 16, mv, 0)

        def scatter(b):
            pltpu.sync_copy(rb[b], acc.at[cidx], add=True)

        start_gather(0, 0)

        def blk(g, carry):
            for b, off in ((1, 1), (0, 2)):
                i = 2 * g + off
                start_gather(i, b)
                fill_cidx(i - 1)
                wait_gather(i - 1, 1 - b)
                scatter(1 - b)
            return carry

        lax.fori_loop(0, nblk, blk, 0)

        fill_cidx(nch - 1)
        wait_gather(nch - 1, 0)
        scatter(0)

        plsc.subcore_barrier()
        pltpu.sync_copy(acc.at[pl.ds(s * _RPT, _RPT)],
                        out.at[pl.ds(c * _NPAD + s * _RPT, _RPT)])

    return k


def _make_deg():
    mesh = plsc.VectorSubcoreMesh(core_axis_name="c", subcore_axis_name="s")

    @functools.partial(
        pl.kernel, mesh=mesh,
        out_type=jax.ShapeDtypeStruct((_NW, _NPAD), jnp.float32),
        scratch_types=[
            pltpu.VMEM((_CHD,), jnp.int32),
            pltpu.VMEM((_NPAD,), jnp.float32),
        ],
        compiler_params=pltpu.CompilerParams(needs_layout_passes=False),
    )
    def k(cols, out, cidx, hist):
        c = lax.axis_index("c")
        s = lax.axis_index("s")
        wid = s * _NC + c

        z16 = jnp.zeros((16,), jnp.float32)

        def zero(i, carry):
            hist[pl.ds(i * 16, 16)] = z16
            return carry

        lax.fori_loop(0, _NPAD // 16, zero, 0)

        def chunk(i, carry):
            base = wid * _EPW + i * _CHD
            pltpu.sync_copy(cols.at[pl.ds(base, _CHD)], cidx)

            def vec(j, carry2):
                idx16 = cidx[pl.ds(j * 16, 16)]
                cnt, last = plsc.scan_count(idx16)
                plsc.addupdate_scatter(
                    hist, [idx16], cnt.astype(jnp.float32), mask=last)
                return carry2

            return lax.fori_loop(0, _CHD // 16, vec, carry)

        lax.fori_loop(0, _NCHUNKS_D, chunk, 0)
        pltpu.sync_copy(hist, out.at[wid])

    return k


_prop128 = _make_prop(_DHID, _CH)
_prop8 = _make_prop(8, 2000, tc_tiling=False)
_deg_pass = _make_deg()


def _dinv_from(deg_part_ref):
    ones32 = jnp.ones((_NW, 1), jnp.float32)
    deg = lax.dot_general(deg_part_ref[...], ones32,
                          (((0,), (0,)), ((), ())),
                          preferred_element_type=jnp.float32)
    return lax.rsqrt(deg[: _N, :] + 1.0)


def _tc1_body(dp, f, w, g1):
    dinv = _dinv_from(dp)
    g1[...] = jnp.dot(f[...], w[...], preferred_element_type=jnp.float32) * dinv


def _tc2_body(dp, scat1, g1, b1, w2p, g2):
    dinv = _dinv_from(dp)
    agg = scat1[: _N, :] + scat1[_NPAD : _NPAD + _N, :] + g1[...]
    x1 = jnp.maximum(agg * dinv + b1[...], 0.0)
    g2[...] = jnp.dot(x1, w2p[...], preferred_element_type=jnp.float32) * dinv


def _tc3_body(dp, scat2, g2, b2p, out):
    dinv = _dinv_from(dp)
    z = (scat2[: _N, :] + scat2[_NPAD : _NPAD + _N, :] + g2[...]) * dinv
    out[...] = z[:, :3] + b2p[...]


def kernel(features, edges, edges2, edge_features, W1, b1, W2, b2):
    del edges2, edge_features
    rows = edges[0]
    cols = edges[1]

    zeros128 = jnp.zeros((_RPT, _DHID), jnp.float32)
    zeros8 = jnp.zeros((_RPT, 8), jnp.float32)
    b1_2d = b1.reshape(1, _DHID)
    b2_2d = b2.reshape(1, 3)
    w2p = jnp.zeros((_DHID, 8), jnp.float32).at[:, :3].set(W2)

    deg_part = _deg_pass(cols)

    g1 = pl.pallas_call(
        _tc1_body,
        out_shape=jax.ShapeDtypeStruct((_N, _DHID), jnp.float32),
    )(deg_part, features, W1)

    scat1 = _prop128(g1, rows, cols, zeros128)

    g2 = pl.pallas_call(
        _tc2_body,
        out_shape=jax.ShapeDtypeStruct((_N, 8), jnp.float32),
    )(deg_part, scat1, g1, b1_2d, w2p)

    scat2 = _prop8(g2, rows, cols, zeros8)

    out = pl.pallas_call(
        _tc3_body,
        out_shape=jax.ShapeDtypeStruct((_N, 3), jnp.float32),
    )(deg_part, scat2, g2, b2_2d)

    return out
